# Initial kernel scaffold; baseline (speedup 1.0000x reference)
import jax
import jax.numpy as jnp
from jax.experimental import pallas as pl


def kernel(user_emb, item_emb, edge_index, edge_weight):
    raise NotImplementedError("write your pallas kernel here")


# speedup vs baseline: 3.5494x; 3.5494x over previous
import functools

import jax
import jax.numpy as jnp
from jax import lax
from jax.experimental import pallas as pl
from jax.experimental.pallas import tpu as pltpu
from jax.experimental.pallas import tpu_sc as plsc

N_USERS = 5000
N_ITEMS = 5000
N = N_USERS + N_ITEMS
EMB = 128
HALF = EMB // 2
E = 320000
LAYERS = 3

NS = 16
EPT = E // NS
CH = 128
NCH = (EPT + CH - 1) // CH
TAIL = EPT - (NCH - 1) * CH
NP = 10240
RPT = NP // NS
ZR = 128


def _body(x0s, src_hbm, dst_hbm, w_hbm, y1, y2, y3,
          src2d, dst2d, w2d, rows, zbuf, acc, gsem):
    c = lax.axis_index("c")
    s = lax.axis_index("s")
    base = s * EPT
    row0 = s * RPT

    zi = jnp.zeros((16,), jnp.int32)
    zf = jnp.zeros((16,), jnp.float32)

    def load_chunk(j, carry):
        off = base + j * CH
        pltpu.sync_copy(src_hbm.at[pl.ds(off, CH)], src2d.at[j])
        pltpu.sync_copy(dst_hbm.at[pl.ds(off, CH)], dst2d.at[j])
        pltpu.sync_copy(w_hbm.at[pl.ds(off, CH)], w2d.at[j])
        return carry
    lax.fori_loop(0, NCH - 1, load_chunk, 0)
    toff = base + (NCH - 1) * CH
    pltpu.sync_copy(src_hbm.at[pl.ds(toff, TAIL)],
                    src2d.at[NCH - 1].at[pl.ds(0, TAIL)])
    pltpu.sync_copy(dst_hbm.at[pl.ds(toff, TAIL)],
                    dst2d.at[NCH - 1].at[pl.ds(0, TAIL)])
    pltpu.sync_copy(w_hbm.at[pl.ds(toff, TAIL)],
                    w2d.at[NCH - 1].at[pl.ds(0, TAIL)])
    for t in range((CH - TAIL) // 16):
        sl = pl.ds(TAIL + t * 16, 16)
        src2d[NCH - 1, sl] = zi
        dst2d[NCH - 1, sl] = zi
        w2d[NCH - 1, sl] = zf

    def zrow(r, carry):
        for k in range(HALF // 16):
            zbuf[r, pl.ds(k * 16, 16)] = zf
        return carry
    lax.fori_loop(0, ZR, zrow, 0)

    srcs = (x0s, y1, y2)
    outs = (y1, y2, y3)
    for L in range(LAYERS):
        xsrc = srcs[L].at[c]
        for k in range(RPT // ZR):
            pltpu.sync_copy(zbuf, acc.at[pl.ds(row0 + k * ZR, ZR)])
        plsc.subcore_barrier()

        def chunk(j, carry):
            pltpu.async_copy(xsrc.at[src2d.at[j]], rows, gsem).wait()
            def scale_group(g, carry2):
                wv16 = w2d[j, pl.ds(g * 16, 16)]
                for r16 in range(16):
                    wv = jnp.full((16,), wv16[r16], jnp.float32)
                    r = g * 16 + r16
                    for k in range(HALF // 16):
                        sl = pl.ds(k * 16, 16)
                        rows[r, sl] = rows[r, sl] * wv
                return carry2
            lax.fori_loop(0, CH // 16, scale_group, 0)
            pltpu.sync_copy(rows, acc.at[dst2d.at[j]], add=True)
            return carry
        lax.fori_loop(0, NCH, chunk, 0)
        plsc.subcore_barrier()

        pltpu.sync_copy(acc.at[pl.ds(row0, RPT)],
                        outs[L].at[c].at[pl.ds(row0, RPT)])
        plsc.subcore_barrier()


def _propagate(x0s, src, dst, w):
    mesh = plsc.VectorSubcoreMesh(core_axis_name="c", subcore_axis_name="s")
    fn = pl.kernel(
        _body,
        out_type=[jax.ShapeDtypeStruct((2, NP, HALF), jnp.float32)] * LAYERS,
        mesh=mesh,
        scratch_types=[
            pltpu.VMEM((NCH, CH), jnp.int32),
            pltpu.VMEM((NCH, CH), jnp.int32),
            pltpu.VMEM((NCH, CH), jnp.float32),
            pltpu.VMEM((CH, HALF), jnp.float32),
            pltpu.VMEM((ZR, HALF), jnp.float32),
            pltpu.VMEM_SHARED((NP, HALF), jnp.float32),
            pltpu.SemaphoreType.DMA,
        ],
        compiler_params=pltpu.CompilerParams(use_tc_tiling_on_sc=False),
    )
    return fn(x0s, src, dst, w)


def kernel(user_emb, item_emb, edge_index, edge_weight):
    x0 = jnp.concatenate([user_emb, item_emb], axis=0)
    x0p = jnp.pad(x0, ((0, NP - N), (0, 0)))
    x0s = jnp.stack([x0p[:, :HALF], x0p[:, HALF:]])
    ys = _propagate(x0s, edge_index[0], edge_index[1], edge_weight)
    layers = [x0] + [jnp.concatenate([y[0, :N], y[1, :N]], axis=-1)
                     for y in ys]
    return jnp.stack(layers)
